# baseline (speedup 1.0000x reference)
import jax
import jax.numpy as jnp
from jax import lax
from jax.experimental import pallas as pl
from jax.experimental.pallas import tpu as pltpu
from jax.experimental.pallas import tpu_sc as plsc

N_ITEMS = 100000
ORDER = 64
PERIOD = 365.25
BATCH = 16384

NC = 2
NS = 16
L = 16
NW = NC * NS
B_PER_W = BATCH // NW
CHUNK = 128
N_CHUNKS = B_PER_W // CHUNK
G_PER_CHUNK = CHUNK // L
N_CHAINS = 4

TWO_PI_OVER_P = float(2.0 * 3.141592653589793 / PERIOD)
TWO_OVER_PI = float(2.0 / 3.141592653589793)
PIO2_HI = 1.57079637050628662109375
PIO2_LO = 4.37113900018624283e-8
MAGIC = 12582912.0


def _cos_sin(theta):
    kf = (theta * TWO_OVER_PI + MAGIC) - MAGIC
    r = (theta - kf * PIO2_HI) + kf * PIO2_LO
    q = kf.astype(jnp.int32) & 3
    r2 = r * r
    sp = r + r * r2 * (-1.6666654611e-1
                       + r2 * (8.3321608736e-3 + r2 * (-1.9515295891e-4)))
    cp = 1.0 - 0.5 * r2 + r2 * r2 * (4.166664568298827e-2
                                     + r2 * (-1.388731625493765e-3
                                             + r2 * 2.443315711809948e-5))
    odd = (q & 1) == 1
    s1 = jnp.where(odd, cp, sp)
    c1 = jnp.where(odd, -sp, cp)
    neg = q >= 2
    s1 = jnp.where(neg, -s1, s1)
    c1 = jnp.where(neg, -c1, c1)
    return c1, s1


def _sc_kernel(t_hbm, off_hbm, idxh_hbm, a_hbm, b_hbm, out_hbm,
               idxh_v, t_v, off_v, a_rows, b_rows, out_v, sem0, sem1):
    wid = lax.axis_index("s") * NC + lax.axis_index("c")
    base = wid * B_PER_W
    sems = (sem0, sem1)

    pltpu.sync_copy(idxh_hbm.at[pl.ds(base, B_PER_W)], idxh_v)
    pltpu.sync_copy(t_hbm.at[pl.ds(base, B_PER_W)], t_v)
    pltpu.sync_copy(off_hbm.at[pl.ds(base, B_PER_W)], off_v)

    handles = {}

    def fire(c, slot):
        isl = pl.ds(c * CHUNK, CHUNK)
        handles[c] = (
            pltpu.async_copy(a_hbm.at[idxh_v.at[isl]], a_rows.at[slot], sems[slot]),
            pltpu.async_copy(b_hbm.at[idxh_v.at[isl]], b_rows.at[slot], sems[slot]),
        )

    fire(0, 0)
    fire(1, 1)

    for c in range(N_CHUNKS):
        slot = c & 1
        for h in handles[c]:
            h.wait()

        def group_body(g, carry, _c=c, _slot=slot):
            gbase = _c * CHUNK + g * L
            tv = t_v[pl.ds(gbase, L)]
            offv = off_v[pl.ds(gbase, L)]
            row = g * L + lax.iota(jnp.int32, L)

            c1, s1 = _cos_sin(tv * TWO_PI_OVER_P)
            c2 = c1 * c1 - s1 * s1
            s2 = 2.0 * c1 * s1
            c3 = c2 * c1 - s2 * s1
            s3 = s2 * c1 + c2 * s1
            c4 = c2 * c2 - s2 * s2
            s4 = 2.0 * c2 * s2

            cn = [c1, c2, c3, c4]
            sn = [s1, s2, s3, s4]
            acc = [jnp.zeros((L,), jnp.float32) for _ in range(N_CHAINS)]
            col = [offv + k for k in range(N_CHAINS)]
            ar = a_rows.at[_slot]
            br = b_rows.at[_slot]
            for j in range(ORDER // N_CHAINS):
                for k in range(N_CHAINS):
                    av = plsc.load_gather(ar, [row, col[k]])
                    bv = plsc.load_gather(br, [row, col[k]])
                    acc[k] = acc[k] + cn[k] * av + sn[k] * bv
                    if j + 1 < ORDER // N_CHAINS:
                        cnk = cn[k] * c4 - sn[k] * s4
                        snk = sn[k] * c4 + cn[k] * s4
                        cn[k], sn[k] = cnk, snk
                        col[k] = col[k] + N_CHAINS
            out_v[pl.ds(gbase, L)] = (acc[0] + acc[1]) + (acc[2] + acc[3])
            return carry

        lax.fori_loop(0, G_PER_CHUNK, group_body, 0)

        if c + 2 < N_CHUNKS:
            fire(c + 2, slot)

    pltpu.sync_copy(out_v, out_hbm.at[pl.ds(base, B_PER_W)])


def kernel(t, item_id, a_table, b_table):
    a50 = a_table.reshape(N_ITEMS // 2, 2 * ORDER)
    b50 = b_table.reshape(N_ITEMS // 2, 2 * ORDER)
    ids = item_id.reshape(-1).astype(jnp.int32)
    idxh = ids >> 1
    off = (ids & 1) << 6
    t1 = t.reshape(-1)

    mesh = plsc.VectorSubcoreMesh(core_axis_name="c", subcore_axis_name="s")
    run = pl.kernel(
        _sc_kernel,
        mesh=mesh,
        compiler_params=pltpu.CompilerParams(needs_layout_passes=False),
        out_type=jax.ShapeDtypeStruct((BATCH,), jnp.float32),
        scratch_types=[
            pltpu.VMEM((B_PER_W,), jnp.int32),
            pltpu.VMEM((B_PER_W,), jnp.float32),
            pltpu.VMEM((B_PER_W,), jnp.int32),
            pltpu.VMEM((2, CHUNK, 2 * ORDER), jnp.float32),
            pltpu.VMEM((2, CHUNK, 2 * ORDER), jnp.float32),
            pltpu.VMEM((B_PER_W,), jnp.float32),
            pltpu.SemaphoreType.DMA,
            pltpu.SemaphoreType.DMA,
        ],
    )
    out = run(t1, off, idxh, a50, b50)
    return out.reshape(BATCH, 1)


# speedup vs baseline: 1.2059x; 1.2059x over previous
import jax
import jax.numpy as jnp
from jax import lax
from jax.experimental import pallas as pl
from jax.experimental.pallas import tpu as pltpu
from jax.experimental.pallas import tpu_sc as plsc

N_ITEMS = 100000
ORDER = 64
PERIOD = 365.25
BATCH = 16384

NC = 2
NS = 16
L = 16
NW = NC * NS
B_PER_W = BATCH // NW
CHUNK = 128
N_CHUNKS = B_PER_W // CHUNK
G_PER_CHUNK = CHUNK // L
N_CHAINS = 4

TWO_PI_OVER_P = float(2.0 * 3.141592653589793 / PERIOD)
TWO_OVER_PI = float(2.0 / 3.141592653589793)
PIO2_HI = 1.57079637050628662109375
PIO2_LO = 4.37113900018624283e-8
MAGIC = 12582912.0


def _cos_sin(theta):
    kf = (theta * TWO_OVER_PI + MAGIC) - MAGIC
    r = (theta - kf * PIO2_HI) + kf * PIO2_LO
    q = kf.astype(jnp.int32) & 3
    r2 = r * r
    sp = r + r * r2 * (-1.6666654611e-1
                       + r2 * (8.3321608736e-3 + r2 * (-1.9515295891e-4)))
    cp = 1.0 - 0.5 * r2 + r2 * r2 * (4.166664568298827e-2
                                     + r2 * (-1.388731625493765e-3
                                             + r2 * 2.443315711809948e-5))
    odd = (q & 1) == 1
    s1 = jnp.where(odd, cp, sp)
    c1 = jnp.where(odd, -sp, cp)
    neg = q >= 2
    s1 = jnp.where(neg, -s1, s1)
    c1 = jnp.where(neg, -c1, c1)
    return c1, s1


def _sc_kernel(t_hbm, ids_hbm, ab_hbm, out_hbm,
               ids_v, t_v, rows, out_v, sem0, sem1):
    wid = lax.axis_index("s") * NC + lax.axis_index("c")
    base = wid * B_PER_W
    sems = (sem0, sem1)

    pltpu.sync_copy(ids_hbm.at[pl.ds(base, B_PER_W)], ids_v)
    pltpu.sync_copy(t_hbm.at[pl.ds(base, B_PER_W)], t_v)

    handles = {}

    def fire(c, slot):
        isl = pl.ds(c * CHUNK, CHUNK)
        handles[c] = pltpu.async_copy(
            ab_hbm.at[ids_v.at[isl]], rows.at[slot], sems[slot])

    fire(0, 0)
    fire(1, 1)

    for c in range(N_CHUNKS):
        slot = c & 1
        handles[c].wait()

        def group_body(g, carry, _c=c, _slot=slot):
            gbase = _c * CHUNK + g * L
            tv = t_v[pl.ds(gbase, L)]
            row = g * L + lax.iota(jnp.int32, L)

            c1, s1 = _cos_sin(tv * TWO_PI_OVER_P)
            c2 = c1 * c1 - s1 * s1
            s2 = 2.0 * c1 * s1
            c3 = c2 * c1 - s2 * s1
            s3 = s2 * c1 + c2 * s1
            c4 = c2 * c2 - s2 * s2
            s4 = 2.0 * c2 * s2

            cn = [c1, c2, c3, c4]
            sn = [s1, s2, s3, s4]
            acc = [jnp.zeros((L,), jnp.float32) for _ in range(N_CHAINS)]
            rr = rows.at[_slot]
            for j in range(ORDER // N_CHAINS):
                for k in range(N_CHAINS):
                    ca = jnp.full((L,), 4 * j + k, jnp.int32)
                    av = plsc.load_gather(rr, [row, ca])
                    bv = plsc.load_gather(rr, [row, ca + ORDER])
                    acc[k] = acc[k] + cn[k] * av + sn[k] * bv
                    if j + 1 < ORDER // N_CHAINS:
                        cnk = cn[k] * c4 - sn[k] * s4
                        snk = sn[k] * c4 + cn[k] * s4
                        cn[k], sn[k] = cnk, snk
            out_v[pl.ds(gbase, L)] = (acc[0] + acc[1]) + (acc[2] + acc[3])
            return carry

        lax.fori_loop(0, G_PER_CHUNK, group_body, 0)

        if c + 2 < N_CHUNKS:
            fire(c + 2, slot)

    pltpu.sync_copy(out_v, out_hbm.at[pl.ds(base, B_PER_W)])


def kernel(t, item_id, a_table, b_table):
    ab = jnp.concatenate([a_table, b_table], axis=1)
    ids = item_id.reshape(-1).astype(jnp.int32)
    t1 = t.reshape(-1)

    mesh = plsc.VectorSubcoreMesh(core_axis_name="c", subcore_axis_name="s")
    run = pl.kernel(
        _sc_kernel,
        mesh=mesh,
        compiler_params=pltpu.CompilerParams(needs_layout_passes=False),
        out_type=jax.ShapeDtypeStruct((BATCH,), jnp.float32),
        scratch_types=[
            pltpu.VMEM((B_PER_W,), jnp.int32),
            pltpu.VMEM((B_PER_W,), jnp.float32),
            pltpu.VMEM((2, CHUNK, 2 * ORDER), jnp.float32),
            pltpu.VMEM((B_PER_W,), jnp.float32),
            pltpu.SemaphoreType.DMA,
            pltpu.SemaphoreType.DMA,
        ],
    )
    out = run(t1, ids, ab)
    return out.reshape(BATCH, 1)


# speedup vs baseline: 1.2074x; 1.0012x over previous
import jax
import jax.numpy as jnp
from jax import lax
from jax.experimental import pallas as pl
from jax.experimental.pallas import tpu as pltpu
from jax.experimental.pallas import tpu_sc as plsc

N_ITEMS = 100000
ORDER = 64
PERIOD = 365.25
BATCH = 16384

NC = 2
NS = 16
L = 16
NW = NC * NS
B_PER_W = BATCH // NW
CHUNK = 128
N_CHUNKS = B_PER_W // CHUNK
G_PER_CHUNK = CHUNK // L
N_CHAINS = 4

TWO_PI_OVER_P = float(2.0 * 3.141592653589793 / PERIOD)
TWO_OVER_PI = float(2.0 / 3.141592653589793)
PIO2_HI = 1.57079637050628662109375
PIO2_LO = 4.37113900018624283e-8
MAGIC = 12582912.0


def _cos_sin(theta):
    kf = (theta * TWO_OVER_PI + MAGIC) - MAGIC
    r = (theta - kf * PIO2_HI) + kf * PIO2_LO
    q = kf.astype(jnp.int32) & 3
    r2 = r * r
    sp = r + r * r2 * (-1.6666654611e-1
                       + r2 * (8.3321608736e-3 + r2 * (-1.9515295891e-4)))
    cp = 1.0 - 0.5 * r2 + r2 * r2 * (4.166664568298827e-2
                                     + r2 * (-1.388731625493765e-3
                                             + r2 * 2.443315711809948e-5))
    odd = (q & 1) == 1
    s1 = jnp.where(odd, cp, sp)
    c1 = jnp.where(odd, -sp, cp)
    neg = q >= 2
    s1 = jnp.where(neg, -s1, s1)
    c1 = jnp.where(neg, -c1, c1)
    return c1, s1


def _sc_kernel(t_hbm, ids_hbm, ab_hbm, out_hbm,
               ids_v, t_v, rows, out_v, sem0, sem1):
    wid = lax.axis_index("s") * NC + lax.axis_index("c")
    base = wid * B_PER_W
    sems = (sem0, sem1)

    pltpu.sync_copy(ids_hbm.at[pl.ds(base, B_PER_W)], ids_v)
    pltpu.sync_copy(t_hbm.at[pl.ds(base, B_PER_W)], t_v)

    handles = {}

    def fire(c, slot):
        isl = pl.ds(c * CHUNK, CHUNK)
        handles[c] = pltpu.async_copy(
            ab_hbm.at[ids_v.at[isl]], rows.at[slot], sems[slot])

    fire(0, 0)
    fire(1, 1)

    for c in range(N_CHUNKS):
        slot = c & 1
        handles[c].wait()

        def group_body(g, carry, _c=c, _slot=slot):
            gbase = _c * CHUNK + g * L
            tv = t_v[pl.ds(gbase, L)]
            row = g * L + lax.iota(jnp.int32, L)

            c1, s1 = _cos_sin(tv * TWO_PI_OVER_P)
            c2 = c1 * c1 - s1 * s1
            s2 = 2.0 * c1 * s1
            c3 = c2 * c1 - s2 * s1
            s3 = s2 * c1 + c2 * s1
            c4 = c2 * c2 - s2 * s2
            s4 = 2.0 * c2 * s2

            cn = [c1, c2, c3, c4]
            sn = [s1, s2, s3, s4]
            acc = [jnp.zeros((L,), jnp.float32) for _ in range(N_CHAINS)]
            rr = rows.at[_slot]
            for j in range(ORDER // N_CHAINS):
                for k in range(N_CHAINS):
                    ca = jnp.full((L,), 4 * j + k, jnp.int32)
                    av = plsc.load_gather(rr, [row, ca])
                    bv = plsc.load_gather(rr, [row, ca + ORDER])
                    acc[k] = acc[k] + cn[k] * av + sn[k] * bv
                    if j + 1 < ORDER // N_CHAINS:
                        cnk = cn[k] * c4 - sn[k] * s4
                        snk = sn[k] * c4 + cn[k] * s4
                        cn[k], sn[k] = cnk, snk
            out_v[pl.ds(gbase, L)] = (acc[0] + acc[1]) + (acc[2] + acc[3])
            return carry

        lax.fori_loop(0, G_PER_CHUNK, group_body, 0)

        if c + 2 < N_CHUNKS:
            fire(c + 2, slot)

    pltpu.sync_copy(out_v, out_hbm.at[pl.ds(base, B_PER_W)])


def kernel(t, item_id, a_table, b_table):
    ab = jnp.concatenate([a_table.T, b_table.T], axis=0).T
    ids = item_id.reshape(-1).astype(jnp.int32)
    t1 = t.reshape(-1)

    mesh = plsc.VectorSubcoreMesh(core_axis_name="c", subcore_axis_name="s")
    run = pl.kernel(
        _sc_kernel,
        mesh=mesh,
        compiler_params=pltpu.CompilerParams(needs_layout_passes=False),
        out_type=jax.ShapeDtypeStruct((BATCH,), jnp.float32),
        scratch_types=[
            pltpu.VMEM((B_PER_W,), jnp.int32),
            pltpu.VMEM((B_PER_W,), jnp.float32),
            pltpu.VMEM((2, CHUNK, 2 * ORDER), jnp.float32),
            pltpu.VMEM((B_PER_W,), jnp.float32),
            pltpu.SemaphoreType.DMA,
            pltpu.SemaphoreType.DMA,
        ],
    )
    out = run(t1, ids, ab)
    return out.reshape(BATCH, 1)


# speedup vs baseline: 1.4591x; 1.2084x over previous
import jax
import jax.numpy as jnp
from jax import lax
from jax.experimental import pallas as pl
from jax.experimental.pallas import tpu as pltpu
from jax.experimental.pallas import tpu_sc as plsc

N_ITEMS = 100000
ORDER = 64
PERIOD = 365.25
BATCH = 16384

NC = 2
NS = 16
L = 16
NGRP = BATCH // L
W = 6272
WLAST = N_ITEMS - 15 * W
HB = 8
NPH = (ORDER // NC) // HB
PAD = BATCH + L

TWO_PI_OVER_P = float(2.0 * 3.141592653589793 / PERIOD)
TWO_OVER_PI = float(2.0 / 3.141592653589793)
PIO2_HI = 1.57079637050628662109375
PIO2_LO = 4.37113900018624283e-8
MAGIC = 12582912.0


def _cos_sin(theta):
    kf = (theta * TWO_OVER_PI + MAGIC) - MAGIC
    r = (theta - kf * PIO2_HI) + kf * PIO2_LO
    q = kf.astype(jnp.int32) & 3
    r2 = r * r
    sp = r + r * r2 * (-1.6666654611e-1
                       + r2 * (8.3321608736e-3 + r2 * (-1.9515295891e-4)))
    cp = 1.0 - 0.5 * r2 + r2 * r2 * (4.166664568298827e-2
                                     + r2 * (-1.388731625493765e-3
                                             + r2 * 2.443315711809948e-5))
    odd = (q & 1) == 1
    s1 = jnp.where(odd, cp, sp)
    c1 = jnp.where(odd, -sp, cp)
    neg = q >= 2
    s1 = jnp.where(neg, -s1, s1)
    c1 = jnp.where(neg, -c1, c1)
    return c1, s1


def _sq(c, s):
    return c * c - s * s, 2.0 * c * s


def _mul(c0, s0, c1, s1):
    return c0 * c1 - s0 * s1, s0 * c1 + c0 * s1


def _sc_kernel(t_hbm, ids_hbm, at_hbm, bt_hbm, tail_hbm, out_hbm,
               ids_v, t_v, pos_v, acc_v, blk, tmp_v, red_v, shared, sem):
    c = lax.axis_index("c")
    s = lax.axis_index("s")

    pltpu.sync_copy(ids_hbm, ids_v.at[pl.ds(0, BATCH)])
    pltpu.sync_copy(t_hbm, t_v.at[pl.ds(0, BATCH)])

    lo = s * W
    wid_s = jnp.where(s == NS - 1, WLAST, W)
    hi = lo + wid_s

    def compact(g, k):
        grp = ids_v[pl.ds(g * L, L)]
        posg = g * L + lax.iota(jnp.int32, L)
        m = (grp >= lo) & (grp < hi)
        plsc.store_compressed(ids_v.at[pl.ds(k, L)], grp - lo, mask=m)
        plsc.store_compressed(pos_v.at[pl.ds(k, L)], posg, mask=m)
        return k + jnp.sum(m.astype(jnp.int32))

    k = lax.fori_loop(0, NGRP, compact, 0)
    ids_v[pl.ds(k, L)] = jnp.zeros((L,), jnp.int32)
    pos_v[pl.ds(k, L)] = jnp.full((L,), BATCH, jnp.int32)
    ngrp = (k + L - 1) // L

    def zero_acc(g, carry):
        acc_v[pl.ds(g * L, L)] = jnp.zeros((L,), jnp.float32)
        return carry
    lax.fori_loop(0, (PAD // L), zero_acc, 0)

    W1 = 46 * 128

    def fetch(tab, tail, p):
        row0 = 32 * c + HB * p
        pltpu.async_copy(
            tab.at[pl.ds(row0, HB), pl.ds(lo, W1)],
            blk.at[:, pl.ds(0, W1)], sem).wait()

        @pl.when(s < NS - 1)
        def _():
            pltpu.async_copy(
                tab.at[pl.ds(row0, HB), pl.ds(lo + W1, W - W1)],
                blk.at[:, pl.ds(W1, W - W1)], sem).wait()

        @pl.when(s == NS - 1)
        def _():
            pltpu.async_copy(
                tail.at[pl.ds(row0, HB), pl.ds(0, 128)],
                blk.at[:, pl.ds(W1, 128)], sem).wait()

    for p in range(NPH):
        for tab_i, tab in ((0, at_hbm), (1, bt_hbm)):
            fetch(tab, tail_hbm, p)

            def phase_body(g, carry, _p=p, _tab_i=tab_i):
                gsl = pl.ds(g * L, L)
                posg = pos_v[gsl]
                colg = ids_v[gsl]
                tv = plsc.load_gather(t_v, [posg])
                c1, s1 = _cos_sin(tv * TWO_PI_OVER_P)
                c2, s2 = _sq(c1, s1)
                c4, s4 = _sq(c2, s2)
                c8, s8 = _sq(c4, s4)
                if _p == 0:
                    cb, sb = c1, s1
                elif _p == 1:
                    cb, sb = _mul(c8, s8, c1, s1)
                elif _p == 2:
                    c16, s16 = _sq(c8, s8)
                    cb, sb = _mul(c16, s16, c1, s1)
                else:
                    c16, s16 = _sq(c8, s8)
                    c24, s24 = _mul(c16, s16, c8, s8)
                    cb, sb = _mul(c24, s24, c1, s1)
                c16b, s16b = _sq(c8, s8)
                c32, s32 = _sq(c16b, s16b)
                cbc, sbc = _mul(cb, sb, c32, s32)
                on1 = c == 1
                cb = jnp.where(on1, cbc, cb)
                sb = jnp.where(on1, sbc, sb)

                acc = acc_v[gsl]
                if _tab_i == 1:
                    tail_lane = (colg >= 5888) & (s == NS - 1)
                    colg = colg + jnp.where(tail_lane, 32, 0)
                cn, sn = cb, sb
                for r in range(HB):
                    rowc = jnp.full((L,), r, jnp.int32)
                    v = plsc.load_gather(blk, [rowc, colg])
                    if _tab_i == 0:
                        acc = acc + cn * v
                    else:
                        acc = acc + sn * v
                    if r + 1 < HB:
                        cn, sn = _mul(cn, sn, c1, s1)
                acc_v[gsl] = acc
                return carry

            lax.fori_loop(0, ngrp, phase_body, 0)

    def zero_out(g, carry):
        t_v[pl.ds(g * L, L)] = jnp.zeros((L,), jnp.float32)
        return carry
    lax.fori_loop(0, (PAD // L), zero_out, 0)

    def scatter(g, carry):
        gsl = pl.ds(g * L, L)
        plsc.store_scatter(t_v, [pos_v[gsl]], acc_v[gsl])
        return carry
    lax.fori_loop(0, ngrp, scatter, 0)

    HALF = BATCH // 2
    seg = HALF // NS
    for h in range(2):
        pltpu.sync_copy(t_v.at[pl.ds(h * HALF, HALF)], shared.at[s])
        plsc.subcore_barrier()

        base = s * seg

        def zero_red(g, carry):
            red_v[pl.ds(g * L, L)] = jnp.zeros((L,), jnp.float32)
            return carry
        lax.fori_loop(0, seg // L, zero_red, 0)

        for j in range(NS):
            pltpu.sync_copy(shared.at[j].at[pl.ds(base, seg)], tmp_v)

            def addup(g, carry):
                gsl = pl.ds(g * L, L)
                red_v[gsl] = red_v[gsl] + tmp_v[gsl]
                return carry
            lax.fori_loop(0, seg // L, addup, 0)

        pltpu.sync_copy(red_v, out_hbm.at[c, pl.ds(h * HALF + base, seg)])
        plsc.subcore_barrier()


def kernel(t, item_id, a_table, b_table):
    at = a_table.T
    bt = b_table.T
    tail = jnp.pad(
        jnp.concatenate(
            [a_table[15 * W + 46 * 128:].T, b_table[15 * W + 46 * 128:].T],
            axis=1),
        ((0, 0), (0, 64)))
    ids = item_id.reshape(-1).astype(jnp.int32)
    t1 = t.reshape(-1)

    mesh = plsc.VectorSubcoreMesh(core_axis_name="c", subcore_axis_name="s")
    run = pl.kernel(
        _sc_kernel,
        mesh=mesh,
        compiler_params=pltpu.CompilerParams(needs_layout_passes=False),
        out_type=jax.ShapeDtypeStruct((NC, BATCH), jnp.float32),
        scratch_types=[
            pltpu.VMEM((PAD,), jnp.int32),
            pltpu.VMEM((PAD,), jnp.float32),
            pltpu.VMEM((PAD,), jnp.int32),
            pltpu.VMEM((PAD,), jnp.float32),
            pltpu.VMEM((HB, W), jnp.float32),
            pltpu.VMEM((BATCH // 2 // NS,), jnp.float32),
            pltpu.VMEM((BATCH // 2 // NS,), jnp.float32),
            pltpu.VMEM_SHARED((NS, BATCH // 2), jnp.float32),
            pltpu.SemaphoreType.DMA,
        ],
    )
    out = run(t1, ids, at, bt, tail)
    return (out[0] + out[1]).reshape(BATCH, 1)


# speedup vs baseline: 1.6743x; 1.1475x over previous
import jax
import jax.numpy as jnp
from jax import lax
from jax.experimental import pallas as pl
from jax.experimental.pallas import tpu as pltpu
from jax.experimental.pallas import tpu_sc as plsc

N_ITEMS = 100000
ORDER = 64
PERIOD = 365.25
BATCH = 16384

NC = 2
NS = 16
L = 16
NGRP = BATCH // L
W = 6272
WLAST = N_ITEMS - 15 * W
HB = 8
NPH = (ORDER // NC) // HB
PAD = BATCH + 2 * L

TWO_PI_OVER_P = float(2.0 * 3.141592653589793 / PERIOD)
TWO_OVER_PI = float(2.0 / 3.141592653589793)
PIO2_HI = 1.57079637050628662109375
PIO2_LO = 4.37113900018624283e-8
MAGIC = 12582912.0


def _cos_sin(theta):
    kf = (theta * TWO_OVER_PI + MAGIC) - MAGIC
    r = (theta - kf * PIO2_HI) + kf * PIO2_LO
    q = kf.astype(jnp.int32) & 3
    r2 = r * r
    sp = r + r * r2 * (-1.6666654611e-1
                       + r2 * (8.3321608736e-3 + r2 * (-1.9515295891e-4)))
    cp = 1.0 - 0.5 * r2 + r2 * r2 * (4.166664568298827e-2
                                     + r2 * (-1.388731625493765e-3
                                             + r2 * 2.443315711809948e-5))
    odd = (q & 1) == 1
    s1 = jnp.where(odd, cp, sp)
    c1 = jnp.where(odd, -sp, cp)
    neg = q >= 2
    s1 = jnp.where(neg, -s1, s1)
    c1 = jnp.where(neg, -c1, c1)
    return c1, s1


def _sq(c, s):
    return c * c - s * s, 2.0 * c * s


def _mul(c0, s0, c1, s1):
    return c0 * c1 - s0 * s1, s0 * c1 + c0 * s1


def _sc_kernel(t_hbm, ids_hbm, at_hbm, bt_hbm, tail_hbm, out_hbm,
               ids_v, t_v, pos_v, acc_v, blk, tmp_v, red_v, shared, sem):
    c = lax.axis_index("c")
    s = lax.axis_index("s")

    pltpu.sync_copy(ids_hbm, ids_v.at[pl.ds(0, BATCH)])
    pltpu.sync_copy(t_hbm, t_v.at[pl.ds(0, BATCH)])

    lo = s * W
    wid_s = jnp.where(s == NS - 1, WLAST, W)
    hi = lo + wid_s

    def compact(g, k):
        grp = ids_v[pl.ds(g * L, L)]
        posg = g * L + lax.iota(jnp.int32, L)
        m = (grp >= lo) & (grp < hi)
        plsc.store_compressed(ids_v.at[pl.ds(k, L)], grp - lo, mask=m)
        plsc.store_compressed(pos_v.at[pl.ds(k, L)], posg, mask=m)
        return k + jnp.sum(m.astype(jnp.int32))

    k = lax.fori_loop(0, NGRP, compact, 0)
    for w in range(2):
        ids_v[pl.ds(k + w * L, L)] = jnp.zeros((L,), jnp.int32)
        pos_v[pl.ds(k + w * L, L)] = jnp.full((L,), BATCH, jnp.int32)
    ngrp2 = (k + 2 * L - 1) // (2 * L)

    def zero_acc(g, carry):
        acc_v[pl.ds(g * L, L)] = jnp.zeros((L,), jnp.float32)
        return carry
    lax.fori_loop(0, (PAD // L), zero_acc, 0)

    W1 = 46 * 128

    def fetch(tab, tail, p):
        row0 = 32 * c + HB * p
        pltpu.async_copy(
            tab.at[pl.ds(row0, HB), pl.ds(lo, W1)],
            blk.at[:, pl.ds(0, W1)], sem).wait()

        @pl.when(s < NS - 1)
        def _():
            pltpu.async_copy(
                tab.at[pl.ds(row0, HB), pl.ds(lo + W1, W - W1)],
                blk.at[:, pl.ds(W1, W - W1)], sem).wait()

        @pl.when(s == NS - 1)
        def _():
            pltpu.async_copy(
                tail.at[pl.ds(row0, HB), pl.ds(0, 128)],
                blk.at[:, pl.ds(W1, 128)], sem).wait()

    for p in range(NPH):
        for tab_i, tab in ((0, at_hbm), (1, bt_hbm)):
            fetch(tab, tail_hbm, p)

            def phase_body(g2, carry, _p=p, _tab_i=tab_i):
                m0 = 32 * c + HB * _p
                fm = (m0 + 1).astype(jnp.float32) * TWO_PI_OVER_P
                st = []
                for u in range(2):
                    gsl = pl.ds((2 * g2 + u) * L, L)
                    posg = pos_v[gsl]
                    colg = ids_v[gsl]
                    if _tab_i == 1:
                        tail_lane = (colg >= 5888) & (s == NS - 1)
                        colg = colg + jnp.where(tail_lane, 32, 0)
                    tv = plsc.load_gather(t_v, [posg])
                    c1, s1 = _cos_sin(tv * TWO_PI_OVER_P)
                    cb, sb = _cos_sin(tv * fm)
                    st.append([gsl, colg, c1, s1, cb, sb, acc_v[gsl]])
                for r in range(HB):
                    rowc = jnp.full((L,), r, jnp.int32)
                    for u in range(2):
                        gsl, colg, c1, s1, cn, sn, acc = st[u]
                        v = plsc.load_gather(blk, [rowc, colg])
                        if _tab_i == 0:
                            acc = acc + cn * v
                        else:
                            acc = acc + sn * v
                        if r + 1 < HB:
                            cn, sn = _mul(cn, sn, c1, s1)
                        st[u] = [gsl, colg, c1, s1, cn, sn, acc]
                for u in range(2):
                    acc_v[st[u][0]] = st[u][6]
                return carry

            lax.fori_loop(0, ngrp2, phase_body, 0)

    def zero_out(g, carry):
        t_v[pl.ds(g * L, L)] = jnp.zeros((L,), jnp.float32)
        return carry
    lax.fori_loop(0, (PAD // L), zero_out, 0)

    def scatter(g, carry):
        gsl = pl.ds(g * L, L)
        plsc.store_scatter(t_v, [pos_v[gsl]], acc_v[gsl])
        return carry
    lax.fori_loop(0, 2 * ngrp2, scatter, 0)

    HALF = BATCH // 2
    seg = HALF // NS
    for h in range(2):
        pltpu.sync_copy(t_v.at[pl.ds(h * HALF, HALF)], shared.at[s])
        plsc.subcore_barrier()

        base = s * seg

        def zero_red(g, carry):
            red_v[pl.ds(g * L, L)] = jnp.zeros((L,), jnp.float32)
            return carry
        lax.fori_loop(0, seg // L, zero_red, 0)

        for j in range(NS):
            pltpu.sync_copy(shared.at[j].at[pl.ds(base, seg)], tmp_v)

            def addup(g, carry):
                gsl = pl.ds(g * L, L)
                red_v[gsl] = red_v[gsl] + tmp_v[gsl]
                return carry
            lax.fori_loop(0, seg // L, addup, 0)

        pltpu.sync_copy(red_v, out_hbm.at[c, pl.ds(h * HALF + base, seg)])
        plsc.subcore_barrier()


def kernel(t, item_id, a_table, b_table):
    at = a_table.T
    bt = b_table.T
    tail = jnp.pad(
        jnp.concatenate(
            [a_table[15 * W + 46 * 128:].T, b_table[15 * W + 46 * 128:].T],
            axis=1),
        ((0, 0), (0, 64)))
    ids = item_id.reshape(-1).astype(jnp.int32)
    t1 = t.reshape(-1)

    mesh = plsc.VectorSubcoreMesh(core_axis_name="c", subcore_axis_name="s")
    run = pl.kernel(
        _sc_kernel,
        mesh=mesh,
        compiler_params=pltpu.CompilerParams(needs_layout_passes=False),
        out_type=jax.ShapeDtypeStruct((NC, BATCH), jnp.float32),
        scratch_types=[
            pltpu.VMEM((PAD,), jnp.int32),
            pltpu.VMEM((PAD,), jnp.float32),
            pltpu.VMEM((PAD,), jnp.int32),
            pltpu.VMEM((PAD,), jnp.float32),
            pltpu.VMEM((HB, W), jnp.float32),
            pltpu.VMEM((BATCH // 2 // NS,), jnp.float32),
            pltpu.VMEM((BATCH // 2 // NS,), jnp.float32),
            pltpu.VMEM_SHARED((NS, BATCH // 2), jnp.float32),
            pltpu.SemaphoreType.DMA,
        ],
    )
    out = run(t1, ids, at, bt, tail)
    return (out[0] + out[1]).reshape(BATCH, 1)


# speedup vs baseline: 1.6828x; 1.0051x over previous
import jax
import jax.numpy as jnp
from jax import lax
from jax.experimental import pallas as pl
from jax.experimental.pallas import tpu as pltpu
from jax.experimental.pallas import tpu_sc as plsc

N_ITEMS = 100000
ORDER = 64
PERIOD = 365.25
BATCH = 16384

NC = 2
NS = 16
L = 16
NGRP = BATCH // L
W = 6272
WLAST = N_ITEMS - 15 * W
HB = 8
NPH = (ORDER // NC) // HB
PAD = BATCH + 3 * L

TWO_PI_OVER_P = float(2.0 * 3.141592653589793 / PERIOD)
TWO_OVER_PI = float(2.0 / 3.141592653589793)
PIO2_HI = 1.57079637050628662109375
PIO2_LO = 4.37113900018624283e-8
MAGIC = 12582912.0


def _cos_sin(theta):
    kf = (theta * TWO_OVER_PI + MAGIC) - MAGIC
    r = (theta - kf * PIO2_HI) + kf * PIO2_LO
    q = kf.astype(jnp.int32) & 3
    r2 = r * r
    sp = r + r * r2 * (-1.6666654611e-1
                       + r2 * (8.3321608736e-3 + r2 * (-1.9515295891e-4)))
    cp = 1.0 - 0.5 * r2 + r2 * r2 * (4.166664568298827e-2
                                     + r2 * (-1.388731625493765e-3
                                             + r2 * 2.443315711809948e-5))
    odd = (q & 1) == 1
    s1 = jnp.where(odd, cp, sp)
    c1 = jnp.where(odd, -sp, cp)
    neg = q >= 2
    s1 = jnp.where(neg, -s1, s1)
    c1 = jnp.where(neg, -c1, c1)
    return c1, s1


def _sq(c, s):
    return c * c - s * s, 2.0 * c * s


def _mul(c0, s0, c1, s1):
    return c0 * c1 - s0 * s1, s0 * c1 + c0 * s1


def _sc_kernel(t_hbm, ids_hbm, at_hbm, bt_hbm, tail_hbm, out_hbm,
               ids_v, t_v, pos_v, acc_v, blk, tmp_v, red_v, shared, sem):
    c = lax.axis_index("c")
    s = lax.axis_index("s")

    pltpu.sync_copy(ids_hbm, ids_v.at[pl.ds(0, BATCH)])
    pltpu.sync_copy(t_hbm, t_v.at[pl.ds(0, BATCH)])

    lo = s * W
    wid_s = jnp.where(s == NS - 1, WLAST, W)
    hi = lo + wid_s

    def compact(g, k):
        grp = ids_v[pl.ds(g * L, L)]
        posg = g * L + lax.iota(jnp.int32, L)
        m = (grp >= lo) & (grp < hi)
        plsc.store_compressed(ids_v.at[pl.ds(k, L)], grp - lo, mask=m)
        plsc.store_compressed(pos_v.at[pl.ds(k, L)], posg, mask=m)
        return k + jnp.sum(m.astype(jnp.int32))

    k = lax.fori_loop(0, NGRP, compact, 0)
    for w in range(3):
        ids_v[pl.ds(k + w * L, L)] = jnp.zeros((L,), jnp.int32)
        pos_v[pl.ds(k + w * L, L)] = jnp.full((L,), BATCH, jnp.int32)
    ngrp2 = (k + 3 * L - 1) // (3 * L)

    def zero_acc(g, carry):
        acc_v[pl.ds(g * L, L)] = jnp.zeros((L,), jnp.float32)
        return carry
    lax.fori_loop(0, (PAD // L), zero_acc, 0)

    W1 = 46 * 128

    def fetch(tab, tail, p):
        row0 = 32 * c + HB * p
        pltpu.async_copy(
            tab.at[pl.ds(row0, HB), pl.ds(lo, W1)],
            blk.at[:, pl.ds(0, W1)], sem).wait()

        @pl.when(s < NS - 1)
        def _():
            pltpu.async_copy(
                tab.at[pl.ds(row0, HB), pl.ds(lo + W1, W - W1)],
                blk.at[:, pl.ds(W1, W - W1)], sem).wait()

        @pl.when(s == NS - 1)
        def _():
            pltpu.async_copy(
                tail.at[pl.ds(row0, HB), pl.ds(0, 128)],
                blk.at[:, pl.ds(W1, 128)], sem).wait()

    for p in range(NPH):
        for tab_i, tab in ((0, at_hbm), (1, bt_hbm)):
            fetch(tab, tail_hbm, p)

            def phase_body(g2, carry, _p=p, _tab_i=tab_i):
                m0 = 32 * c + HB * _p
                fm = (m0 + 1).astype(jnp.float32) * TWO_PI_OVER_P
                st = []
                for u in range(3):
                    gsl = pl.ds((3 * g2 + u) * L, L)
                    posg = pos_v[gsl]
                    colg = ids_v[gsl]
                    if _tab_i == 1:
                        tail_lane = (colg >= 5888) & (s == NS - 1)
                        colg = colg + jnp.where(tail_lane, 32, 0)
                    tv = plsc.load_gather(t_v, [posg])
                    c1, s1 = _cos_sin(tv * TWO_PI_OVER_P)
                    cb, sb = _cos_sin(tv * fm)
                    st.append([gsl, colg, c1, s1, cb, sb, acc_v[gsl]])
                for r in range(HB):
                    rowc = jnp.full((L,), r, jnp.int32)
                    for u in range(3):
                        gsl, colg, c1, s1, cn, sn, acc = st[u]
                        v = plsc.load_gather(blk, [rowc, colg])
                        if _tab_i == 0:
                            acc = acc + cn * v
                        else:
                            acc = acc + sn * v
                        if r + 1 < HB:
                            cn, sn = _mul(cn, sn, c1, s1)
                        st[u] = [gsl, colg, c1, s1, cn, sn, acc]
                for u in range(3):
                    acc_v[st[u][0]] = st[u][6]
                return carry

            lax.fori_loop(0, ngrp2, phase_body, 0)

    def zero_out(g, carry):
        t_v[pl.ds(g * L, L)] = jnp.zeros((L,), jnp.float32)
        return carry
    lax.fori_loop(0, (PAD // L), zero_out, 0)

    def scatter(g, carry):
        gsl = pl.ds(g * L, L)
        plsc.store_scatter(t_v, [pos_v[gsl]], acc_v[gsl])
        return carry
    lax.fori_loop(0, 3 * ngrp2, scatter, 0)

    HALF = BATCH // 2
    seg = HALF // NS
    for h in range(2):
        pltpu.sync_copy(t_v.at[pl.ds(h * HALF, HALF)], shared.at[s])
        plsc.subcore_barrier()

        base = s * seg

        def zero_red(g, carry):
            red_v[pl.ds(g * L, L)] = jnp.zeros((L,), jnp.float32)
            return carry
        lax.fori_loop(0, seg // L, zero_red, 0)

        for j in range(NS):
            pltpu.sync_copy(shared.at[j].at[pl.ds(base, seg)], tmp_v)

            def addup(g, carry):
                gsl = pl.ds(g * L, L)
                red_v[gsl] = red_v[gsl] + tmp_v[gsl]
                return carry
            lax.fori_loop(0, seg // L, addup, 0)

        pltpu.sync_copy(red_v, out_hbm.at[c, pl.ds(h * HALF + base, seg)])
        plsc.subcore_barrier()


def kernel(t, item_id, a_table, b_table):
    at = a_table.T
    bt = b_table.T
    tail = jnp.pad(
        jnp.concatenate(
            [a_table[15 * W + 46 * 128:].T, b_table[15 * W + 46 * 128:].T],
            axis=1),
        ((0, 0), (0, 64)))
    ids = item_id.reshape(-1).astype(jnp.int32)
    t1 = t.reshape(-1)

    mesh = plsc.VectorSubcoreMesh(core_axis_name="c", subcore_axis_name="s")
    run = pl.kernel(
        _sc_kernel,
        mesh=mesh,
        compiler_params=pltpu.CompilerParams(needs_layout_passes=False),
        out_type=jax.ShapeDtypeStruct((NC, BATCH), jnp.float32),
        scratch_types=[
            pltpu.VMEM((PAD,), jnp.int32),
            pltpu.VMEM((PAD,), jnp.float32),
            pltpu.VMEM((PAD,), jnp.int32),
            pltpu.VMEM((PAD,), jnp.float32),
            pltpu.VMEM((HB, W), jnp.float32),
            pltpu.VMEM((BATCH // 2 // NS,), jnp.float32),
            pltpu.VMEM((BATCH // 2 // NS,), jnp.float32),
            pltpu.VMEM_SHARED((NS, BATCH // 2), jnp.float32),
            pltpu.SemaphoreType.DMA,
        ],
    )
    out = run(t1, ids, at, bt, tail)
    return (out[0] + out[1]).reshape(BATCH, 1)
